# Initial kernel scaffold; baseline (speedup 1.0000x reference)
import jax
import jax.numpy as jnp
from jax.experimental import pallas as pl


def kernel(indices, positions, W):
    raise NotImplementedError("write your pallas kernel here")


# speedup vs baseline: 1.4482x; 1.4482x over previous
import functools
import math

import jax
import jax.numpy as jnp
from jax import lax
from jax.experimental import pallas as pl
from jax.experimental.pallas import tpu as pltpu
from jax.experimental.pallas import tpu_sc as plsc

EMBED_DIM = 64
NUM_CHARS = 95
BATCH = 1024
SEQ = 50
CPAD = 96
ROWS = CPAD * SEQ
ROW_W = EMBED_DIM * 4
N_IDX = BATCH * SEQ


def _table_kernel(wp_ref, pos_ref, idx_ref, table_ref, comb_ref):
    r_c = lax.broadcasted_iota(jnp.int32, (ROWS, CPAD), 0) // SEQ
    c_c = lax.broadcasted_iota(jnp.int32, (ROWS, CPAD), 1)
    onehot_c = jnp.where(r_c == c_c, 1.0, 0.0)
    params = jnp.dot(onehot_c, wp_ref[...],
                     preferred_element_type=jnp.float32)

    r_l = lax.broadcasted_iota(jnp.int32, (ROWS, SEQ), 0) % SEQ
    c_l = lax.broadcasted_iota(jnp.int32, (ROWS, SEQ), 1)
    onehot_l = jnp.where(r_l == c_l, 1.0, 0.0)
    pos_col = pos_ref[...].astype(jnp.float32).reshape(SEQ, 1)
    pos_v = jnp.dot(onehot_l, pos_col,
                    preferred_element_type=jnp.float32)

    omega = params[:, 0:1] * 2.0
    a1 = params[:, 1:2]
    a2 = params[:, 2:3]
    a3 = params[:, 3:4]
    beta = params[:, 4:5]
    gamma = 1.0 / (1.0 + jnp.exp(-params[:, 5:6]))
    phi = params[:, 6:7] * math.pi

    j = lax.broadcasted_iota(jnp.int32, (ROWS, EMBED_DIM), 1).astype(jnp.float32)
    wave = (a1 * jnp.sin(omega * j + phi)
            + a2 * jnp.sin(2.0 * omega * j + 2.0 * phi)
            + a3 * jnp.sin(3.0 * omega * j + 3.0 * phi))
    wave = wave * jnp.exp(-gamma * j)
    wave = wave + beta * j * jnp.sin(pos_v * (0.1 * math.pi))

    p = jnp.concatenate([wave, jnp.sin(wave), jnp.cos(wave)], axis=1)
    rr = lax.broadcasted_iota(jnp.int32, (3 * EMBED_DIM, ROW_W), 0)
    cc = lax.broadcasted_iota(jnp.int32, (3 * EMBED_DIM, ROW_W), 1)
    d = rr % EMBED_DIM
    blk = rr // EMBED_DIM
    sel = ((blk == 0) & (cc == 4 * d)
           | (blk == 0) & (cc == 4 * ((d + 1) % EMBED_DIM) + 1)
           | (blk == 1) & (cc == 4 * d + 2)
           | (blk == 2) & (cc == 4 * d + 3))
    perm = jnp.where(sel, 1.0, 0.0)
    table_ref[...] = jnp.dot(p, perm, preferred_element_type=jnp.float32)

    l_iota = lax.broadcasted_iota(jnp.int32, (BATCH, SEQ), 1)
    comb_ref[...] = idx_ref[...] * SEQ + l_iota


_table_call = pl.pallas_call(
    _table_kernel,
    out_shape=(
        jax.ShapeDtypeStruct((ROWS, ROW_W), jnp.float32),
        jax.ShapeDtypeStruct((BATCH, SEQ), jnp.int32),
    ),
)

try:
    _info = plsc.get_sparse_core_info()
    _NC, _NS = _info.num_cores, _info.num_subcores
except Exception:
    _NC, _NS = 2, 16
_NW = _NC * _NS
_PER_W = N_IDX // _NW
_CHUNK = 80
_N_CHUNK = _PER_W // _CHUNK


def _gather_kernel(table_hbm, comb_hbm, out_hbm, idx_v, rows_v, sem):
    wid = lax.axis_index("s") * _NC + lax.axis_index("c")
    base = wid * _PER_W

    def body(i, carry):
        c0 = base + i * _CHUNK
        pltpu.sync_copy(comb_hbm.at[pl.ds(c0, _CHUNK)], idx_v)
        pltpu.async_copy(table_hbm.at[idx_v], rows_v, sem).wait()
        pltpu.sync_copy(rows_v, out_hbm.at[pl.ds(c0, _CHUNK)])
        return carry

    lax.fori_loop(0, _N_CHUNK, body, 0)


@functools.cache
def _gather_call():
    return pl.kernel(
        _gather_kernel,
        out_type=jax.ShapeDtypeStruct((N_IDX, ROW_W), jnp.float32),
        mesh=plsc.VectorSubcoreMesh(core_axis_name="c", subcore_axis_name="s"),
        scratch_types=[
            pltpu.VMEM((_CHUNK,), jnp.int32),
            pltpu.VMEM((_CHUNK, ROW_W), jnp.float32),
            pltpu.SemaphoreType.DMA,
        ],
    )


def kernel(indices, positions, W):
    wp = jnp.concatenate([W, jnp.zeros((CPAD - NUM_CHARS, 8), W.dtype)], axis=0)
    table, comb = _table_call(wp, positions.reshape(1, SEQ), indices)
    out = _gather_call()(table, comb.reshape(N_IDX))
    return out.reshape(BATCH, SEQ, EMBED_DIM, 4)


# speedup vs baseline: 1.6728x; 1.1551x over previous
import functools
import math

import jax
import jax.numpy as jnp
from jax import lax
from jax.experimental import pallas as pl
from jax.experimental.pallas import tpu as pltpu
from jax.experimental.pallas import tpu_sc as plsc

EMBED_DIM = 64
NUM_CHARS = 95
BATCH = 1024
SEQ = 50
CPAD = 96
ROWS = CPAD * SEQ
ROW_W = EMBED_DIM * 4
N_IDX = BATCH * SEQ


def _table_kernel(wp_ref, pos_ref, idx_ref, table_ref, comb_ref):
    wp = wp_ref[...]
    omega = wp[:, 0:1] * 2.0
    a1 = wp[:, 1:2]
    a2 = wp[:, 2:3]
    a3 = wp[:, 3:4]
    beta = wp[:, 4:5]
    gamma = 1.0 / (1.0 + jnp.exp(-wp[:, 5:6]))
    phi = wp[:, 6:7] * math.pi

    j = lax.broadcasted_iota(jnp.int32, (CPAD, EMBED_DIM), 1).astype(jnp.float32)
    theta = omega * j + phi
    s1 = jnp.sin(theta)
    c1 = jnp.cos(theta)
    base = (a1 * s1 + a2 * (2.0 * s1 * c1)
            + a3 * (3.0 - 4.0 * s1 * s1) * s1) * jnp.exp(-gamma * j)
    betaj = beta * j

    r_c = lax.broadcasted_iota(jnp.int32, (ROWS, CPAD), 0) // SEQ
    c_c = lax.broadcasted_iota(jnp.int32, (ROWS, CPAD), 1)
    onehot_c = jnp.where(r_c == c_c, 1.0, 0.0)
    base_cl = jnp.dot(onehot_c, base, preferred_element_type=jnp.float32)
    betaj_cl = jnp.dot(onehot_c, betaj, preferred_element_type=jnp.float32)

    r_l = lax.broadcasted_iota(jnp.int32, (ROWS, SEQ), 0) % SEQ
    c_l = lax.broadcasted_iota(jnp.int32, (ROWS, SEQ), 1)
    onehot_l = jnp.where(r_l == c_l, 1.0, 0.0)
    ps50 = jnp.sin(pos_ref[...].astype(jnp.float32).reshape(SEQ, 1)
                   * (0.1 * math.pi))
    ps = jnp.dot(onehot_l, ps50, preferred_element_type=jnp.float32)

    wave = base_cl + betaj_cl * ps

    p = jnp.concatenate([wave, jnp.sin(wave), jnp.cos(wave)], axis=1)
    rr = lax.broadcasted_iota(jnp.int32, (3 * EMBED_DIM, ROW_W), 0)
    cc = lax.broadcasted_iota(jnp.int32, (3 * EMBED_DIM, ROW_W), 1)
    d = rr % EMBED_DIM
    blk = rr // EMBED_DIM
    sel = ((blk == 0) & (cc == 4 * d)
           | (blk == 0) & (cc == 4 * ((d + 1) % EMBED_DIM) + 1)
           | (blk == 1) & (cc == 4 * d + 2)
           | (blk == 2) & (cc == 4 * d + 3))
    perm = jnp.where(sel, 1.0, 0.0)
    table_ref[...] = jnp.dot(p, perm, preferred_element_type=jnp.float32)

    l_iota = lax.broadcasted_iota(jnp.int32, (BATCH, SEQ), 1)
    comb_ref[...] = idx_ref[...] * SEQ + l_iota


_table_call = pl.pallas_call(
    _table_kernel,
    out_shape=(
        jax.ShapeDtypeStruct((ROWS, ROW_W), jnp.float32),
        jax.ShapeDtypeStruct((BATCH, SEQ), jnp.int32),
    ),
)

try:
    _info = plsc.get_sparse_core_info()
    _NC, _NS = _info.num_cores, _info.num_subcores
except Exception:
    _NC, _NS = 2, 16
_NW = _NC * _NS
_PER_W = N_IDX // _NW
_CHUNK = 80
_N_CHUNK = _PER_W // _CHUNK


def _gather_kernel(table_hbm, comb_hbm, out_hbm,
                   idx_v, rows0, rows1, sem0, sem1):
    wid = lax.axis_index("s") * _NC + lax.axis_index("c")
    base = wid * _PER_W
    pltpu.sync_copy(comb_hbm.at[pl.ds(base, _PER_W)], idx_v)

    rows = (rows0, rows1)
    sems = (sem0, sem1)

    def start(cp, b):
        pltpu.async_copy(
            table_hbm.at[idx_v.at[pl.ds(cp * _CHUNK, _CHUNK)]], rows[b],
            sems[b])

    start(0, 0)
    start(1, 1)

    def body(g, carry):
        for b in range(2):
            cp = g * 2 + b
            pltpu.make_async_copy(
                table_hbm.at[idx_v.at[pl.ds(cp * _CHUNK, _CHUNK)]], rows[b],
                sems[b]).wait()
            pltpu.sync_copy(rows[b],
                            out_hbm.at[pl.ds(base + cp * _CHUNK, _CHUNK)])

            @pl.when(cp + 2 < _N_CHUNK)
            def _():
                start(cp + 2, b)
        return carry

    lax.fori_loop(0, _N_CHUNK // 2, body, 0)


@functools.cache
def _gather_call():
    return pl.kernel(
        _gather_kernel,
        out_type=jax.ShapeDtypeStruct((N_IDX, ROW_W), jnp.float32),
        mesh=plsc.VectorSubcoreMesh(core_axis_name="c", subcore_axis_name="s"),
        scratch_types=[
            pltpu.VMEM((_PER_W,), jnp.int32),
            pltpu.VMEM((_CHUNK, ROW_W), jnp.float32),
            pltpu.VMEM((_CHUNK, ROW_W), jnp.float32),
            pltpu.SemaphoreType.DMA,
            pltpu.SemaphoreType.DMA,
        ],
    )


def kernel(indices, positions, W):
    wp = jnp.concatenate([W, jnp.zeros((CPAD - NUM_CHARS, 8), W.dtype)], axis=0)
    table, comb = _table_call(wp, positions.reshape(1, SEQ), indices)
    out = _gather_call()(table, comb.reshape(N_IDX))
    return out.reshape(BATCH, SEQ, EMBED_DIM, 4)


# speedup vs baseline: 3.0031x; 1.7953x over previous
import math

import jax
import jax.numpy as jnp
from jax import lax
from jax.experimental import pallas as pl
from jax.experimental.pallas import tpu as pltpu

EMBED_DIM = 64
NUM_CHARS = 95
BATCH = 1024
SEQ = 50
CPAD = 96


def _wave_kernel(pos_ref, idxt_ref, wt_ref, w_ref, r_ref, s_ref, c_ref):
    idx = idxt_ref[0]
    cc = lax.broadcasted_iota(jnp.int32, (CPAD, BATCH), 0)
    onehot = jnp.where(cc == idx, 1.0, 0.0)
    p = jnp.dot(wt_ref[...], onehot,
                preferred_element_type=jnp.float32)
    omega = p[0:1, :] * 2.0
    a1 = p[1:2, :]
    a2 = p[2:3, :]
    a3 = p[3:4, :]
    beta = p[4:5, :]
    gamma = 1.0 / (1.0 + jnp.exp(-p[5:6, :]))
    phi = p[6:7, :] * math.pi

    j = lax.broadcasted_iota(jnp.int32, (EMBED_DIM, BATCH), 0).astype(
        jnp.float32)
    theta = omega * j + phi
    s1 = jnp.sin(theta)
    c1 = jnp.cos(theta)
    base = (a1 * s1 + a2 * (2.0 * s1 * c1)
            + a3 * (3.0 - 4.0 * s1 * s1) * s1) * jnp.exp(-gamma * j)
    psl = jnp.sin(pos_ref[pl.program_id(0)].astype(jnp.float32)
                  * (0.1 * math.pi))
    wave = base + (beta * psl) * j

    w_ref[...] = wave[None]
    r_ref[...] = pltpu.roll(wave, 1, axis=0)[None]
    s_ref[...] = jnp.sin(wave)[None]
    c_ref[...] = jnp.cos(wave)[None]


_wave_call = pl.pallas_call(
    _wave_kernel,
    grid=(SEQ,),
    in_specs=[
        pl.BlockSpec(memory_space=pltpu.SMEM),
        pl.BlockSpec((1, 1, BATCH), lambda l: (l, 0, 0)),
        pl.BlockSpec((8, CPAD), lambda l: (0, 0)),
    ],
    out_specs=[pl.BlockSpec((1, EMBED_DIM, BATCH), lambda l: (l, 0, 0))] * 4,
    out_shape=[jax.ShapeDtypeStruct((SEQ, EMBED_DIM, BATCH), jnp.float32)] * 4,
)


def kernel(indices, positions, W):
    wt = jnp.concatenate(
        [W, jnp.zeros((CPAD - NUM_CHARS, 8), W.dtype)], axis=0).T
    wt_, rt_, st_, ct_ = _wave_call(
        positions, indices.T.reshape(SEQ, 1, BATCH), wt)
    psi = [a.transpose(2, 0, 1) for a in (wt_, rt_, st_, ct_)]
    return jnp.stack(psi, axis=-1)
